# Initial kernel scaffold; baseline (speedup 1.0000x reference)
import jax
import jax.numpy as jnp
from jax.experimental import pallas as pl


def kernel(edge_index, Y, X, alp, lam):
    raise NotImplementedError("write your pallas kernel here")


# speedup vs baseline: 12.5883x; 12.5883x over previous
import functools

import jax
import jax.numpy as jnp
from jax import lax
from jax.experimental import pallas as pl
from jax.experimental.pallas import tpu as pltpu
from jax.experimental.pallas import tpu_sc as plsc

N_NODES = 10000
N_EDGES = 320000
D = 128

NC = 2
NS = 16
NW = NC * NS

CHUNK = 128
CH_PER_W = 80
E_PER_W = CHUNK * CH_PER_W
E_PAD = E_PER_W * NW
N_CHUNK_ROWS = E_PAD // CHUNK

HB = 10240
HB_ROWS = HB // D
AGG_ROWS = 10240
WB_PER_TILE = AGG_ROWS // NS
DH = 64

RB = 400
GRID = N_NODES // RB

_mesh = plsc.VectorSubcoreMesh(core_axis_name="c", subcore_axis_name="s")

_f32 = jnp.float32
_i32 = jnp.int32


def _hist_body(dst2d_hbm, out_hbm, dstv, hist1d, hist2d, rowids, shared):
    c = lax.axis_index("c")
    s = lax.axis_index("s")
    w = c * NS + s

    zeros16 = jnp.zeros((16,), _f32)
    ones16 = jnp.ones((16,), _f32)

    def _z1(i, carry):
        hist1d[pl.ds(i * 16, 16)] = zeros16
        return carry

    lax.fori_loop(0, HB // 16, _z1, 0)

    def _z2(r, carry):
        for cb in range(8):
            hist2d[r, pl.ds(cb * 16, 16)] = zeros16
        return carry

    lax.fori_loop(0, HB_ROWS, _z2, 0)

    @pl.when(s == 0)
    def _():
        pltpu.sync_copy(hist2d, shared)

    for i in range(HB_ROWS // 16):
        rowids[pl.ds(i * 16, 16)] = lax.iota(_i32, 16) + i * 16

    plsc.subcore_barrier()

    pltpu.sync_copy(dst2d_hbm.at[pl.ds(w * CH_PER_W, CH_PER_W)], dstv)

    def _hrow(r, carry):
        for cb in range(8):
            idx = dstv[r, pl.ds(cb * 16, 16)]
            plsc.addupdate_scatter(hist1d, [idx], ones16)
        return carry

    lax.fori_loop(0, CH_PER_W, _hrow, 0)

    def _pack(r, carry):
        for cb in range(8):
            hist2d[r, pl.ds(cb * 16, 16)] = hist1d[pl.ds(r * D + cb * 16, 16)]
        return carry

    lax.fori_loop(0, HB_ROWS, _pack, 0)

    pltpu.sync_copy(hist2d, shared.at[rowids], add=True)
    plsc.subcore_barrier()

    @pl.when(s < HB_ROWS // 8)
    def _():
        pltpu.sync_copy(shared.at[pl.ds(s * 8, 8)], out_hbm.at[c, pl.ds(s * 8, 8)])


_hist = functools.partial(
    pl.kernel,
    out_type=jax.ShapeDtypeStruct((NC, HB_ROWS, D), _f32),
    mesh=_mesh,
    scratch_types=[
        pltpu.VMEM((CH_PER_W, CHUNK), _i32),
        pltpu.VMEM((HB,), _f32),
        pltpu.VMEM((HB_ROWS, D), _f32),
        pltpu.VMEM((HB_ROWS,), _i32),
        pltpu.VMEM_SHARED((HB_ROWS, D), _f32),
    ],
    compiler_params=pltpu.CompilerParams(needs_layout_passes=False),
)(_hist_body)


def _prep_body(degp_ref, y_ref, ys0_ref, ys1_ref, dinv_ref):
    deg = degp_ref[0] + degp_ref[1]
    dinv = jnp.where(deg > 0, lax.rsqrt(jnp.maximum(deg, 1.0)), 0.0)
    dinv_ref[...] = dinv
    ys0_ref[...] = y_ref[:, :DH] * dinv
    ys1_ref[...] = y_ref[:, DH:] * dinv


_prep = pl.pallas_call(
    _prep_body,
    grid=(GRID,),
    in_specs=[
        pl.BlockSpec((2, RB, 1), lambda i: (0, i, 0)),
        pl.BlockSpec((RB, D), lambda i: (i, 0)),
    ],
    out_specs=[
        pl.BlockSpec((RB, DH), lambda i: (i, 0)),
        pl.BlockSpec((RB, DH), lambda i: (i, 0)),
        pl.BlockSpec((RB, 1), lambda i: (i, 0)),
    ],
    out_shape=[
        jax.ShapeDtypeStruct((N_NODES, DH), _f32),
        jax.ShapeDtypeStruct((N_NODES, DH), _f32),
        jax.ShapeDtypeStruct((N_NODES, 1), _f32),
    ],
)


def _agg_body(src2d_hbm, dst2d_hbm, ys_hbm, out_hbm,
              sidx, didx, rows_a, rows_b, sem_a, sem_b, agg):
    c = lax.axis_index("c")
    s = lax.axis_index("s")
    w = c * NS + s

    zeros16 = jnp.zeros((16,), _f32)

    def _zrow(r, carry):
        for cb in range(DH // 16):
            rows_a[r, pl.ds(cb * 16, 16)] = zeros16
        return carry

    lax.fori_loop(0, CHUNK, _zrow, 0)
    for k in range(AGG_ROWS // NS // CHUNK):
        pltpu.sync_copy(rows_a, agg.at[pl.ds(s * (AGG_ROWS // NS) + k * CHUNK, CHUNK)])
    plsc.subcore_barrier()

    pltpu.sync_copy(src2d_hbm.at[pl.ds(w * CH_PER_W, CH_PER_W)], sidx)
    pltpu.sync_copy(dst2d_hbm.at[pl.ds(w * CH_PER_W, CH_PER_W)], didx)

    def _pair(k, carry):
        j0 = 2 * k
        cp_a = pltpu.async_copy(ys_hbm.at[sidx.at[j0]], rows_a, sem_a)
        cp_b = pltpu.async_copy(ys_hbm.at[sidx.at[j0 + 1]], rows_b, sem_b)
        cp_a.wait()
        pltpu.sync_copy(rows_a, agg.at[didx.at[j0]], add=True)
        cp_b.wait()
        pltpu.sync_copy(rows_b, agg.at[didx.at[j0 + 1]], add=True)
        return carry

    lax.fori_loop(0, CH_PER_W // 2, _pair, 0)
    plsc.subcore_barrier()

    pltpu.sync_copy(agg.at[pl.ds(s * WB_PER_TILE, WB_PER_TILE)],
                    out_hbm.at[c, pl.ds(s * WB_PER_TILE, WB_PER_TILE)])


_agg = functools.partial(
    pl.kernel,
    out_type=jax.ShapeDtypeStruct((NC, AGG_ROWS, DH), _f32),
    mesh=_mesh,
    scratch_types=[
        pltpu.VMEM((CH_PER_W, CHUNK), _i32),
        pltpu.VMEM((CH_PER_W, CHUNK), _i32),
        pltpu.VMEM((CHUNK, DH), _f32),
        pltpu.VMEM((CHUNK, DH), _f32),
        pltpu.SemaphoreType.DMA,
        pltpu.SemaphoreType.DMA,
        pltpu.VMEM_SHARED((AGG_ROWS, DH), _f32),
    ],
    compiler_params=pltpu.CompilerParams(needs_layout_passes=False,
                                         use_tc_tiling_on_sc=False),
)(_agg_body)


def _comb_body(coef_ref, y_ref, x_ref, dinv_ref, ap0_ref, ap1_ref, out_ref):
    c0, c1, c2 = coef_ref[0], coef_ref[1], coef_ref[2]
    dinv = dinv_ref[...]
    out_ref[:, :DH] = (c0 * y_ref[:, :DH]
                       + c1 * (dinv * (ap0_ref[0] + ap0_ref[1]))
                       + c2 * x_ref[:, :DH])
    out_ref[:, DH:] = (c0 * y_ref[:, DH:]
                       + c1 * (dinv * (ap1_ref[0] + ap1_ref[1]))
                       + c2 * x_ref[:, DH:])


_comb = pl.pallas_call(
    _comb_body,
    grid=(GRID,),
    in_specs=[
        pl.BlockSpec(memory_space=pltpu.MemorySpace.SMEM),
        pl.BlockSpec((RB, D), lambda i: (i, 0)),
        pl.BlockSpec((RB, D), lambda i: (i, 0)),
        pl.BlockSpec((RB, 1), lambda i: (i, 0)),
        pl.BlockSpec((2, RB, DH), lambda i: (0, i, 0)),
        pl.BlockSpec((2, RB, DH), lambda i: (0, i, 0)),
    ],
    out_specs=pl.BlockSpec((RB, D), lambda i: (i, 0)),
    out_shape=jax.ShapeDtypeStruct((N_NODES, D), _f32),
)


def kernel(edge_index, Y, X, alp, lam):
    src = edge_index[0]
    dst = edge_index[1]

    npad = E_PAD - N_EDGES
    ar = jnp.arange(npad, dtype=_i32)
    src_p = jnp.concatenate([src, ar % N_NODES]).reshape(N_CHUNK_ROWS, CHUNK)
    dst_p = jnp.concatenate([dst, N_NODES + ar % (AGG_ROWS - N_NODES)]
                            ).reshape(N_CHUNK_ROWS, CHUNK)

    degp = _hist(dst_p)
    ys0, ys1, dinv = _prep(degp.reshape(NC, HB, 1)[:, :N_NODES], Y)

    aggp0 = _agg(src_p, dst_p, ys0)
    aggp1 = _agg(src_p, dst_p, ys1)

    al = alp * lam
    coef = jnp.stack([1.0 - al - alp, al, alp]).astype(_f32)
    return _comb(coef, Y, X, dinv, aggp0, aggp1)


# speedup vs baseline: 13.7617x; 1.0932x over previous
import functools

import jax
import jax.numpy as jnp
from jax import lax
from jax.experimental import pallas as pl
from jax.experimental.pallas import tpu as pltpu
from jax.experimental.pallas import tpu_sc as plsc

N_NODES = 10000
N_EDGES = 320000
D = 128

NC = 2
NS = 16
NW = NC * NS

CHUNK = 128
N_CHUNK_ROWS = 2560
E_PAD = N_CHUNK_ROWS * CHUNK
CH_T = N_CHUNK_ROWS // NS
NPAIR = CH_T // 2

HB = 10240
HB_ROWS = HB // D
AGG_ROWS = 10240
WB_PER_TILE = AGG_ROWS // NS
DH = 64
RS = N_NODES // NS
RQ = 125

RB = 400
GRID = N_NODES // RB

_mesh = plsc.VectorSubcoreMesh(core_axis_name="c", subcore_axis_name="s")

_f32 = jnp.float32
_i32 = jnp.int32


def _fused_body(dst2d_hbm, src2d_hbm, y0_hbm, y1_hbm,
                aggh_hbm, dinv_hbm, ys0_hbm, ys1_hbm,
                dstv, sidx, hist1d, hist2d, rowids, histv, dinvv, dinv_vm,
                rows_a, rows_b, sem_a, sem_b, hist_sh, agg):
    c = lax.axis_index("c")
    s = lax.axis_index("s")

    zeros16 = jnp.zeros((16,), _f32)
    ones16 = jnp.ones((16,), _f32)

    def _z1(i, carry):
        hist1d[pl.ds(i * 16, 16)] = zeros16
        return carry

    lax.fori_loop(0, HB // 16, _z1, 0)

    def _z2(r, carry):
        for cb in range(8):
            hist2d[r, pl.ds(cb * 16, 16)] = zeros16
        return carry

    lax.fori_loop(0, HB_ROWS, _z2, 0)

    def _z3(r, carry):
        for cb in range(DH // 16):
            rows_a[r, pl.ds(cb * 16, 16)] = zeros16
        return carry

    lax.fori_loop(0, CHUNK, _z3, 0)

    for i in range(HB_ROWS // 16):
        rowids[pl.ds(i * 16, 16)] = lax.iota(_i32, 16) + i * 16

    for k in range(WB_PER_TILE // CHUNK):
        pltpu.sync_copy(rows_a, agg.at[pl.ds(s * WB_PER_TILE + k * CHUNK, CHUNK)])

    @pl.when(s == 0)
    def _():
        pltpu.sync_copy(hist2d, hist_sh)

    plsc.subcore_barrier()

    pltpu.sync_copy(dst2d_hbm.at[pl.ds(s * CH_T, CH_T)], dstv)

    def _hrow(r, carry):
        for cb in range(8):
            idx = dstv[r, pl.ds(cb * 16, 16)]
            plsc.addupdate_scatter(hist1d, [idx], ones16)
        return carry

    lax.fori_loop(0, CH_T, _hrow, 0)

    def _pack(r, carry):
        for cb in range(8):
            hist2d[r, pl.ds(cb * 16, 16)] = hist1d[pl.ds(r * D + cb * 16, 16)]
        return carry

    lax.fori_loop(0, HB_ROWS, _pack, 0)

    pltpu.sync_copy(hist2d, hist_sh.at[rowids], add=True)
    plsc.subcore_barrier()

    pltpu.sync_copy(hist_sh.at[pl.ds(s * 5, 5)], histv)
    half16 = jnp.full((16,), 0.5, _f32)
    t32 = jnp.full((16,), 1.5, _f32)
    magic = jnp.full((16,), 0x5F3759DF, _i32)
    for r in range(5):
        for cb in range(8):
            x = histv[r, pl.ds(cb * 16, 16)]
            yv = plsc.bitcast(magic - (plsc.bitcast(x, _i32) >> 1), _f32)
            for _ in range(3):
                yv = yv * (t32 - half16 * x * yv * yv)
            dinvv[r, pl.ds(cb * 16, 16)] = jnp.where(x > 0, yv, 0.0)
    pltpu.sync_copy(dinvv, hist_sh.at[pl.ds(s * 5, 5)])

    @pl.when(c == 0)
    def _():
        pltpu.sync_copy(dinvv, dinv_hbm.at[pl.ds(s * 5, 5)])

    plsc.subcore_barrier()
    pltpu.sync_copy(hist_sh, hist2d)

    def _unpack(r, carry):
        for cb in range(8):
            dinv_vm[pl.ds(r * D + cb * 16, 16)] = hist2d[r, pl.ds(cb * 16, 16)]
        return carry

    lax.fori_loop(0, HB_ROWS, _unpack, 0)

    def _scale(y_hbm, ys_hbm):
        def _q(q, carry):
            gbase = s * RS + q * RQ

            pltpu.sync_copy(y_hbm.at[pl.ds(gbase, RQ)], rows_a.at[pl.ds(0, RQ)])

            def _r(r, carry2):
                gi = jnp.full((16,), gbase + r, _i32)
                dv = plsc.load_gather(dinv_vm, [gi])
                for cb in range(DH // 16):
                    rows_a[r, pl.ds(cb * 16, 16)] = rows_a[r, pl.ds(cb * 16, 16)] * dv
                return carry2

            lax.fori_loop(0, RQ, _r, 0)
            pltpu.sync_copy(rows_a.at[pl.ds(0, RQ)], ys_hbm.at[pl.ds(gbase, RQ)])
            return carry

        lax.fori_loop(0, RS // RQ, _q, 0)

    @pl.when(c == 0)
    def _():
        _scale(y0_hbm, ys0_hbm)

    @pl.when(c == 1)
    def _():
        _scale(y1_hbm, ys1_hbm)

    plsc.subcore_barrier()

    pltpu.sync_copy(src2d_hbm.at[pl.ds(s * CH_T, CH_T)], sidx)

    def _agg_loop(ys_hbm):
        def _pair(k, carry):
            j0 = 2 * k
            cpa = pltpu.async_copy(ys_hbm.at[sidx.at[j0]], rows_a, sem_a)
            cpb = pltpu.async_copy(ys_hbm.at[sidx.at[j0 + 1]], rows_b, sem_b)
            cpa.wait()
            pltpu.sync_copy(rows_a, agg.at[dstv.at[j0]], add=True)
            cpb.wait()
            pltpu.sync_copy(rows_b, agg.at[dstv.at[j0 + 1]], add=True)
            return carry

        lax.fori_loop(0, NPAIR, _pair, 0)

    @pl.when(c == 0)
    def _():
        _agg_loop(ys0_hbm)

    @pl.when(c == 1)
    def _():
        _agg_loop(ys1_hbm)

    plsc.subcore_barrier()

    pltpu.sync_copy(agg.at[pl.ds(s * WB_PER_TILE, WB_PER_TILE)],
                    aggh_hbm.at[c, pl.ds(s * WB_PER_TILE, WB_PER_TILE)])


_fused = functools.partial(
    pl.kernel,
    out_type=(
        jax.ShapeDtypeStruct((NC, AGG_ROWS, DH), _f32),
        jax.ShapeDtypeStruct((HB_ROWS, D), _f32),
        jax.ShapeDtypeStruct((N_NODES, DH), _f32),
        jax.ShapeDtypeStruct((N_NODES, DH), _f32),
    ),
    mesh=_mesh,
    scratch_types=[
        pltpu.VMEM((CH_T, CHUNK), _i32),
        pltpu.VMEM((CH_T, CHUNK), _i32),
        pltpu.VMEM((HB,), _f32),
        pltpu.VMEM((HB_ROWS, D), _f32),
        pltpu.VMEM((HB_ROWS,), _i32),
        pltpu.VMEM((5, D), _f32),
        pltpu.VMEM((5, D), _f32),
        pltpu.VMEM((HB,), _f32),
        pltpu.VMEM((CHUNK, DH), _f32),
        pltpu.VMEM((CHUNK, DH), _f32),
        pltpu.SemaphoreType.DMA,
        pltpu.SemaphoreType.DMA,
        pltpu.VMEM_SHARED((HB_ROWS, D), _f32),
        pltpu.VMEM_SHARED((AGG_ROWS, DH), _f32),
    ],
    compiler_params=pltpu.CompilerParams(needs_layout_passes=False,
                                         use_tc_tiling_on_sc=False),
)(_fused_body)


def _comb_body(coef_ref, y_ref, x_ref, dinv_ref, ap_ref, out_ref):
    c0, c1, c2 = coef_ref[0], coef_ref[1], coef_ref[2]
    dinv = dinv_ref[...]
    out_ref[:, :DH] = (c0 * y_ref[:, :DH]
                       + c1 * (dinv * ap_ref[0])
                       + c2 * x_ref[:, :DH])
    out_ref[:, DH:] = (c0 * y_ref[:, DH:]
                       + c1 * (dinv * ap_ref[1])
                       + c2 * x_ref[:, DH:])


_comb = pl.pallas_call(
    _comb_body,
    grid=(GRID,),
    in_specs=[
        pl.BlockSpec(memory_space=pltpu.MemorySpace.SMEM),
        pl.BlockSpec((RB, D), lambda i: (i, 0)),
        pl.BlockSpec((RB, D), lambda i: (i, 0)),
        pl.BlockSpec((RB, 1), lambda i: (i, 0)),
        pl.BlockSpec((2, RB, DH), lambda i: (0, i, 0)),
    ],
    out_specs=pl.BlockSpec((RB, D), lambda i: (i, 0)),
    out_shape=jax.ShapeDtypeStruct((N_NODES, D), _f32),
)


def kernel(edge_index, Y, X, alp, lam):
    src = edge_index[0]
    dst = edge_index[1]

    npad = E_PAD - N_EDGES
    ar = jnp.arange(npad, dtype=_i32)
    src_p = jnp.concatenate([src, ar % N_NODES]).reshape(N_CHUNK_ROWS, CHUNK)
    dst_p = jnp.concatenate([dst, N_NODES + ar % (AGG_ROWS - N_NODES)]
                            ).reshape(N_CHUNK_ROWS, CHUNK)

    aggh, dinv2d, _ys0, _ys1 = _fused(dst_p, src_p, Y[:, :DH], Y[:, DH:])
    dinv_col = dinv2d.reshape(HB)[:N_NODES].reshape(N_NODES, 1)

    al = alp * lam
    coef = jnp.stack([1.0 - al - alp, al, alp]).astype(_f32)
    return _comb(coef, Y, X, dinv_col, aggh)


# speedup vs baseline: 14.9513x; 1.0864x over previous
import functools

import jax
import jax.numpy as jnp
from jax import lax
from jax.experimental import pallas as pl
from jax.experimental.pallas import tpu as pltpu
from jax.experimental.pallas import tpu_sc as plsc

N_NODES = 10000
N_EDGES = 320000
D = 128

NC = 2
NS = 16
NW = NC * NS

CHUNK = 128
N_CHUNK_ROWS = 2560
E_PAD = N_CHUNK_ROWS * CHUNK
CH_T = N_CHUNK_ROWS // NS
CH_H = CH_T // 2

HB = 10240
HB_ROWS = HB // D
AGG_ROWS = 10240
WB_PER_TILE = AGG_ROWS // NS
DH = 64
RS = N_NODES // NS
RQ = 125

RB = 400
GRID = N_NODES // RB

_mesh = plsc.VectorSubcoreMesh(core_axis_name="c", subcore_axis_name="s")

_f32 = jnp.float32
_i32 = jnp.int32


def _fused_body(dst2d_hbm, src2d_hbm, y0_hbm, y1_hbm,
                aggh_hbm, dinv_hbm, ys0_hbm, ys1_hbm,
                dstv, sidx, hist1d, hist2d, rowids, histv, dinvv, dinv_vm,
                rows_a, rows_b, rows_c, rows_d, sem_a, sem_b, sem_c, sem_d,
                sem_e, sem_f, sem_g, sem_h, hist_sh, agg):
    c = lax.axis_index("c")
    s = lax.axis_index("s")

    zeros16 = jnp.zeros((16,), _f32)
    ones16 = jnp.ones((16,), _f32)

    def _z1(i, carry):
        hist1d[pl.ds(i * 16, 16)] = zeros16
        return carry

    lax.fori_loop(0, HB // 16, _z1, 0)

    def _z2(r, carry):
        for cb in range(8):
            hist2d[r, pl.ds(cb * 16, 16)] = zeros16
        return carry

    lax.fori_loop(0, HB_ROWS, _z2, 0)

    def _z3(r, carry):
        for cb in range(DH // 16):
            rows_a[r, pl.ds(cb * 16, 16)] = zeros16
        return carry

    lax.fori_loop(0, CHUNK, _z3, 0)

    for i in range(HB_ROWS // 16):
        rowids[pl.ds(i * 16, 16)] = lax.iota(_i32, 16) + i * 16

    for k in range(WB_PER_TILE // CHUNK):
        pltpu.sync_copy(rows_a, agg.at[pl.ds(s * WB_PER_TILE + k * CHUNK, CHUNK)])

    @pl.when(s == 0)
    def _():
        pltpu.sync_copy(hist2d, hist_sh)

    plsc.subcore_barrier()

    def _hrow(r, carry):
        for cb in range(8):
            idx = dstv[r, pl.ds(cb * 16, 16)]
            plsc.addupdate_scatter(hist1d, [idx], ones16)
        return carry

    for h in range(2):
        pltpu.sync_copy(dst2d_hbm.at[pl.ds(s * CH_T + h * CH_H, CH_H)], dstv)
        lax.fori_loop(0, CH_H, _hrow, 0)

    def _pack(r, carry):
        for cb in range(8):
            hist2d[r, pl.ds(cb * 16, 16)] = hist1d[pl.ds(r * D + cb * 16, 16)]
        return carry

    lax.fori_loop(0, HB_ROWS, _pack, 0)

    pltpu.sync_copy(hist2d, hist_sh.at[rowids], add=True)
    plsc.subcore_barrier()

    pltpu.sync_copy(hist_sh.at[pl.ds(s * 5, 5)], histv)
    half16 = jnp.full((16,), 0.5, _f32)
    t32 = jnp.full((16,), 1.5, _f32)
    magic = jnp.full((16,), 0x5F3759DF, _i32)
    for r in range(5):
        for cb in range(8):
            x = histv[r, pl.ds(cb * 16, 16)]
            yv = plsc.bitcast(magic - (plsc.bitcast(x, _i32) >> 1), _f32)
            for _ in range(3):
                yv = yv * (t32 - half16 * x * yv * yv)
            dinvv[r, pl.ds(cb * 16, 16)] = jnp.where(x > 0, yv, 0.0)
    pltpu.sync_copy(dinvv, hist_sh.at[pl.ds(s * 5, 5)])

    @pl.when(c == 0)
    def _():
        pltpu.sync_copy(dinvv, dinv_hbm.at[pl.ds(s * 5, 5)])

    plsc.subcore_barrier()
    pltpu.sync_copy(hist_sh, hist2d)

    def _unpack(r, carry):
        for cb in range(8):
            dinv_vm[pl.ds(r * D + cb * 16, 16)] = hist2d[r, pl.ds(cb * 16, 16)]
        return carry

    lax.fori_loop(0, HB_ROWS, _unpack, 0)

    def _scale(y_hbm, ys_hbm):
        def _q(q, carry):
            gbase = s * RS + q * RQ

            pltpu.sync_copy(y_hbm.at[pl.ds(gbase, RQ)], rows_a.at[pl.ds(0, RQ)])

            def _r(r, carry2):
                gi = jnp.full((16,), gbase + r, _i32)
                dv = plsc.load_gather(dinv_vm, [gi])
                for cb in range(DH // 16):
                    rows_a[r, pl.ds(cb * 16, 16)] = rows_a[r, pl.ds(cb * 16, 16)] * dv
                return carry2

            lax.fori_loop(0, RQ, _r, 0)
            pltpu.sync_copy(rows_a.at[pl.ds(0, RQ)], ys_hbm.at[pl.ds(gbase, RQ)])
            return carry

        lax.fori_loop(0, RS // RQ, _q, 0)

    @pl.when(c == 0)
    def _():
        _scale(y0_hbm, ys0_hbm)

    @pl.when(c == 1)
    def _():
        _scale(y1_hbm, ys1_hbm)

    plsc.subcore_barrier()

    def _agg_loop(ys_hbm):
        bufs = (rows_a, rows_b, rows_c, rows_d)
        gsems = (sem_a, sem_b, sem_c, sem_d)
        ssems = (sem_e, sem_f, sem_g, sem_h)

        def _quad(k, carry):
            j0 = 4 * k
            gs = [pltpu.async_copy(ys_hbm.at[sidx.at[j0 + i]], bufs[i], gsems[i])
                  for i in range(4)]
            ss = []
            for i in range(4):
                gs[i].wait()
                ss.append(pltpu.async_copy(bufs[i], agg.at[dstv.at[j0 + i]],
                                           ssems[i], add=True))
            for i in range(4):
                ss[i].wait()
            return carry

        for h in range(2):
            pltpu.sync_copy(src2d_hbm.at[pl.ds(s * CH_T + h * CH_H, CH_H)], sidx)
            pltpu.sync_copy(dst2d_hbm.at[pl.ds(s * CH_T + h * CH_H, CH_H)], dstv)
            lax.fori_loop(0, CH_H // 4, _quad, 0)

    @pl.when(c == 0)
    def _():
        _agg_loop(ys0_hbm)

    @pl.when(c == 1)
    def _():
        _agg_loop(ys1_hbm)

    plsc.subcore_barrier()

    pltpu.sync_copy(agg.at[pl.ds(s * WB_PER_TILE, WB_PER_TILE)],
                    aggh_hbm.at[c, pl.ds(s * WB_PER_TILE, WB_PER_TILE)])


_fused = functools.partial(
    pl.kernel,
    out_type=(
        jax.ShapeDtypeStruct((NC, AGG_ROWS, DH), _f32),
        jax.ShapeDtypeStruct((HB_ROWS, D), _f32),
        jax.ShapeDtypeStruct((N_NODES, DH), _f32),
        jax.ShapeDtypeStruct((N_NODES, DH), _f32),
    ),
    mesh=_mesh,
    scratch_types=[
        pltpu.VMEM((CH_H, CHUNK), _i32),
        pltpu.VMEM((CH_H, CHUNK), _i32),
        pltpu.VMEM((HB,), _f32),
        pltpu.VMEM((HB_ROWS, D), _f32),
        pltpu.VMEM((HB_ROWS,), _i32),
        pltpu.VMEM((5, D), _f32),
        pltpu.VMEM((5, D), _f32),
        pltpu.VMEM((HB,), _f32),
        pltpu.VMEM((CHUNK, DH), _f32),
        pltpu.VMEM((CHUNK, DH), _f32),
        pltpu.VMEM((CHUNK, DH), _f32),
        pltpu.VMEM((CHUNK, DH), _f32),
        pltpu.SemaphoreType.DMA,
        pltpu.SemaphoreType.DMA,
        pltpu.SemaphoreType.DMA,
        pltpu.SemaphoreType.DMA,
        pltpu.SemaphoreType.DMA,
        pltpu.SemaphoreType.DMA,
        pltpu.SemaphoreType.DMA,
        pltpu.SemaphoreType.DMA,
        pltpu.VMEM_SHARED((HB_ROWS, D), _f32),
        pltpu.VMEM_SHARED((AGG_ROWS, DH), _f32),
    ],
    compiler_params=pltpu.CompilerParams(needs_layout_passes=False,
                                         use_tc_tiling_on_sc=False),
)(_fused_body)


def _comb_body(coef_ref, y_ref, x_ref, dinv_ref, ap_ref, out_ref):
    c0, c1, c2 = coef_ref[0], coef_ref[1], coef_ref[2]
    dinv = dinv_ref[...]
    out_ref[:, :DH] = (c0 * y_ref[:, :DH]
                       + c1 * (dinv * ap_ref[0])
                       + c2 * x_ref[:, :DH])
    out_ref[:, DH:] = (c0 * y_ref[:, DH:]
                       + c1 * (dinv * ap_ref[1])
                       + c2 * x_ref[:, DH:])


_comb = pl.pallas_call(
    _comb_body,
    grid=(GRID,),
    in_specs=[
        pl.BlockSpec(memory_space=pltpu.MemorySpace.SMEM),
        pl.BlockSpec((RB, D), lambda i: (i, 0)),
        pl.BlockSpec((RB, D), lambda i: (i, 0)),
        pl.BlockSpec((RB, 1), lambda i: (i, 0)),
        pl.BlockSpec((2, RB, DH), lambda i: (0, i, 0)),
    ],
    out_specs=pl.BlockSpec((RB, D), lambda i: (i, 0)),
    out_shape=jax.ShapeDtypeStruct((N_NODES, D), _f32),
)


def kernel(edge_index, Y, X, alp, lam):
    src = edge_index[0]
    dst = edge_index[1]

    npad = E_PAD - N_EDGES
    ar = jnp.arange(npad, dtype=_i32)
    src_p = jnp.concatenate([src, ar % N_NODES]).reshape(N_CHUNK_ROWS, CHUNK)
    dst_p = jnp.concatenate([dst, N_NODES + ar % (AGG_ROWS - N_NODES)]
                            ).reshape(N_CHUNK_ROWS, CHUNK)

    aggh, dinv2d, _ys0, _ys1 = _fused(dst_p, src_p, Y[:, :DH], Y[:, DH:])
    dinv_col = dinv2d.reshape(HB)[:N_NODES].reshape(N_NODES, 1)

    al = alp * lam
    coef = jnp.stack([1.0 - al - alp, al, alp]).astype(_f32)
    return _comb(coef, Y, X, dinv_col, aggh)


# speedup vs baseline: 15.8826x; 1.0623x over previous
import functools

import jax
import jax.numpy as jnp
from jax import lax
from jax.experimental import pallas as pl
from jax.experimental.pallas import tpu as pltpu
from jax.experimental.pallas import tpu_sc as plsc

N_NODES = 10000
N_EDGES = 320000
D = 128

NC = 2
NS = 16
NW = NC * NS

CHUNK = 64
E_PAD = 327680
N_CHUNK_ROWS = E_PAD // CHUNK
CH_T = N_CHUNK_ROWS // NS
CH_Q = CH_T // 4

HB = 10240
HB_ROWS = HB // D
AGG_ROWS = 10240
SHARE = AGG_ROWS // NS
DH = 64

RB = 400
GRID = N_NODES // RB

_mesh = plsc.VectorSubcoreMesh(core_axis_name="c", subcore_axis_name="s")

_f32 = jnp.float32
_i32 = jnp.int32


def _fused_body(dst2d_hbm, src2d_hbm, y0_hbm, y1_hbm,
                aggh_hbm, dinv_hbm,
                dstv, sidx, hist1d, packb, rowids, histv, dinvv, dinv_lv,
                rows_a, rows_b, rows_c, rows_d, sem_a, sem_b, sem_c, sem_d,
                sem_e, sem_f, sem_g, sem_h, hist_sh, ys_sp, agg):
    c = lax.axis_index("c")
    s = lax.axis_index("s")

    zeros16 = jnp.zeros((16,), _f32)
    ones16 = jnp.ones((16,), _f32)

    def _z1(i, carry):
        hist1d[pl.ds(i * 16, 16)] = zeros16
        return carry

    lax.fori_loop(0, HB // 16, _z1, 0)

    def _zp(r, carry):
        for cb in range(8):
            packb[r, pl.ds(cb * 16, 16)] = zeros16
        return carry

    lax.fori_loop(0, HB_ROWS // 4, _zp, 0)

    def _za(r, carry):
        for cb in range(DH // 16):
            rows_a[r, pl.ds(cb * 16, 16)] = zeros16
        return carry

    lax.fori_loop(0, CHUNK, _za, 0)

    for p in range(4):
        rowids[p, pl.ds(0, 16)] = lax.iota(_i32, 16) + p * 20
        rowids[p, pl.ds(4, 16)] = lax.iota(_i32, 16) + (p * 20 + 4)

    for k in range(SHARE // CHUNK):
        pltpu.sync_copy(rows_a, agg.at[pl.ds(s * SHARE + k * CHUNK, CHUNK)])

    @pl.when(s == 0)
    def _():
        for p in range(4):
            pltpu.sync_copy(packb, hist_sh.at[pl.ds(p * 20, 20)])

    plsc.subcore_barrier()

    def _hrow(r, carry):
        for cb in range(CHUNK // 16):
            idx = dstv[r, pl.ds(cb * 16, 16)]
            plsc.addupdate_scatter(hist1d, [idx], ones16)
        return carry

    for h in range(4):
        pltpu.sync_copy(dst2d_hbm.at[pl.ds(s * CH_T + h * CH_Q, CH_Q)], dstv)
        lax.fori_loop(0, CH_Q, _hrow, 0)

    for p in range(4):
        def _pk(r, carry, p=p):
            for cb in range(8):
                packb[r, pl.ds(cb * 16, 16)] = \
                    hist1d[pl.ds((p * 20 + r) * D + cb * 16, 16)]
            return carry

        lax.fori_loop(0, HB_ROWS // 4, _pk, 0)
        pltpu.sync_copy(packb, hist_sh.at[rowids.at[p]], add=True)

    plsc.subcore_barrier()

    pltpu.sync_copy(hist_sh.at[pl.ds(s * 5, 5)], histv)
    half16 = jnp.full((16,), 0.5, _f32)
    t32 = jnp.full((16,), 1.5, _f32)
    magic = jnp.full((16,), 0x5F3759DF, _i32)
    for r in range(5):
        for cb in range(8):
            x = histv[r, pl.ds(cb * 16, 16)]
            yv = plsc.bitcast(magic - (plsc.bitcast(x, _i32) >> 1), _f32)
            for _ in range(3):
                yv = yv * (t32 - half16 * x * yv * yv)
            dv = jnp.where(x > 0, yv, 0.0)
            dinvv[r, pl.ds(cb * 16, 16)] = dv
            dinv_lv[pl.ds((r * 8 + cb) * 16, 16)] = dv

    @pl.when(c == 0)
    def _():
        pltpu.sync_copy(dinvv, dinv_hbm.at[pl.ds(s * 5, 5)])

    def _scale(y_hbm):
        def _q(q, carry):
            base = s * SHARE + q * CHUNK

            pltpu.sync_copy(y_hbm.at[pl.ds(base, CHUNK)], rows_a)

            def _r(r, carry2):
                gi = jnp.full((16,), q * CHUNK + r, _i32)
                dv = plsc.load_gather(dinv_lv, [gi])
                for cb in range(DH // 16):
                    rows_a[r, pl.ds(cb * 16, 16)] = rows_a[r, pl.ds(cb * 16, 16)] * dv
                return carry2

            lax.fori_loop(0, CHUNK, _r, 0)
            pltpu.sync_copy(rows_a, ys_sp.at[pl.ds(base, CHUNK)])
            return carry

        lax.fori_loop(0, SHARE // CHUNK, _q, 0)

    @pl.when(c == 0)
    def _():
        _scale(y0_hbm)

    @pl.when(c == 1)
    def _():
        _scale(y1_hbm)

    plsc.subcore_barrier()

    bufs = (rows_a, rows_b, rows_c, rows_d)
    gsems = (sem_a, sem_b, sem_c, sem_d)
    ssems = (sem_e, sem_f, sem_g, sem_h)

    def _quad(k, carry):
        j0 = 4 * k
        gs = [pltpu.async_copy(ys_sp.at[sidx.at[j0 + i]], bufs[i], gsems[i])
              for i in range(4)]
        ss = []
        for i in range(4):
            gs[i].wait()
            ss.append(pltpu.async_copy(bufs[i], agg.at[dstv.at[j0 + i]],
                                       ssems[i], add=True))
        for i in range(4):
            ss[i].wait()
        return carry

    for h in range(4):
        pltpu.sync_copy(src2d_hbm.at[pl.ds(s * CH_T + h * CH_Q, CH_Q)], sidx)
        pltpu.sync_copy(dst2d_hbm.at[pl.ds(s * CH_T + h * CH_Q, CH_Q)], dstv)
        lax.fori_loop(0, CH_Q // 4, _quad, 0)

    plsc.subcore_barrier()

    pltpu.sync_copy(agg.at[pl.ds(s * SHARE, SHARE)],
                    aggh_hbm.at[c, pl.ds(s * SHARE, SHARE)])


_fused = functools.partial(
    pl.kernel,
    out_type=(
        jax.ShapeDtypeStruct((NC, AGG_ROWS, DH), _f32),
        jax.ShapeDtypeStruct((HB_ROWS, D), _f32),
    ),
    mesh=_mesh,
    scratch_types=[
        pltpu.VMEM((CH_Q, CHUNK), _i32),
        pltpu.VMEM((CH_Q, CHUNK), _i32),
        pltpu.VMEM((HB,), _f32),
        pltpu.VMEM((HB_ROWS // 4, D), _f32),
        pltpu.VMEM((4, 20), _i32),
        pltpu.VMEM((5, D), _f32),
        pltpu.VMEM((5, D), _f32),
        pltpu.VMEM((SHARE,), _f32),
        pltpu.VMEM((CHUNK, DH), _f32),
        pltpu.VMEM((CHUNK, DH), _f32),
        pltpu.VMEM((CHUNK, DH), _f32),
        pltpu.VMEM((CHUNK, DH), _f32),
        pltpu.SemaphoreType.DMA,
        pltpu.SemaphoreType.DMA,
        pltpu.SemaphoreType.DMA,
        pltpu.SemaphoreType.DMA,
        pltpu.SemaphoreType.DMA,
        pltpu.SemaphoreType.DMA,
        pltpu.SemaphoreType.DMA,
        pltpu.SemaphoreType.DMA,
        pltpu.VMEM_SHARED((HB_ROWS, D), _f32),
        pltpu.VMEM_SHARED((AGG_ROWS, DH), _f32),
        pltpu.VMEM_SHARED((AGG_ROWS, DH), _f32),
    ],
    compiler_params=pltpu.CompilerParams(needs_layout_passes=False,
                                         use_tc_tiling_on_sc=False),
)(_fused_body)


def _comb_body(coef_ref, y_ref, x_ref, dinv_ref, ap_ref, out_ref):
    c0, c1, c2 = coef_ref[0], coef_ref[1], coef_ref[2]
    dinv = dinv_ref[...]
    out_ref[:, :DH] = (c0 * y_ref[:, :DH]
                       + c1 * (dinv * ap_ref[0])
                       + c2 * x_ref[:, :DH])
    out_ref[:, DH:] = (c0 * y_ref[:, DH:]
                       + c1 * (dinv * ap_ref[1])
                       + c2 * x_ref[:, DH:])


_comb = pl.pallas_call(
    _comb_body,
    grid=(GRID,),
    in_specs=[
        pl.BlockSpec(memory_space=pltpu.MemorySpace.SMEM),
        pl.BlockSpec((RB, D), lambda i: (i, 0)),
        pl.BlockSpec((RB, D), lambda i: (i, 0)),
        pl.BlockSpec((RB, 1), lambda i: (i, 0)),
        pl.BlockSpec((2, RB, DH), lambda i: (0, i, 0)),
    ],
    out_specs=pl.BlockSpec((RB, D), lambda i: (i, 0)),
    out_shape=jax.ShapeDtypeStruct((N_NODES, D), _f32),
)


def kernel(edge_index, Y, X, alp, lam):
    src = edge_index[0]
    dst = edge_index[1]

    npad = E_PAD - N_EDGES
    ar = jnp.arange(npad, dtype=_i32)
    src_p = jnp.concatenate([src, ar % N_NODES]).reshape(N_CHUNK_ROWS, CHUNK)
    dst_p = jnp.concatenate([dst, N_NODES + ar % (AGG_ROWS - N_NODES)]
                            ).reshape(N_CHUNK_ROWS, CHUNK)

    zpad = jnp.zeros((AGG_ROWS - N_NODES, DH), _f32)
    y0 = jnp.concatenate([Y[:, :DH], zpad])
    y1 = jnp.concatenate([Y[:, DH:], zpad])

    aggh, dinv2d = _fused(dst_p, src_p, y0, y1)
    dinv_col = dinv2d.reshape(HB)[:N_NODES].reshape(N_NODES, 1)

    al = alp * lam
    coef = jnp.stack([1.0 - al - alp, al, alp]).astype(_f32)
    return _comb(coef, Y, X, dinv_col, aggh)


# speedup vs baseline: 15.9639x; 1.0051x over previous
import functools

import jax
import jax.numpy as jnp
from jax import lax
from jax.experimental import pallas as pl
from jax.experimental.pallas import tpu as pltpu
from jax.experimental.pallas import tpu_sc as plsc

N_NODES = 10000
N_EDGES = 320000
D = 128

NC = 2
NS = 16
NW = NC * NS

CHUNK = 64
E_PAD = 327680
N_CHUNK_ROWS = E_PAD // CHUNK
CH_T = N_CHUNK_ROWS // NS
CH_Q = CH_T // 4

HB = 10240
HB_ROWS = HB // D
AGG_ROWS = 10240
SHARE = AGG_ROWS // NS
DH = 64

RB = 400
GRID = N_NODES // RB

_mesh = plsc.VectorSubcoreMesh(core_axis_name="c", subcore_axis_name="s")

_f32 = jnp.float32
_i32 = jnp.int32


def _fused_body(dst2d_hbm, src2d_hbm, y0_hbm, y1_hbm,
                aggh_hbm, dinv_hbm,
                dstv, sidx, hist1d, packb, rowids, histv, dinvv, dinv_lv,
                dumidx,
                rows_a, rows_b, rows_c, rows_d, sem_a, sem_b, sem_c, sem_d,
                sem_e, sem_f, sem_g, sem_h, hist_sh, ys_sp, agg):
    c = lax.axis_index("c")
    s = lax.axis_index("s")

    zeros16 = jnp.zeros((16,), _f32)
    ones16 = jnp.ones((16,), _f32)

    def _z1(i, carry):
        hist1d[pl.ds(i * 16, 16)] = zeros16
        return carry

    lax.fori_loop(0, HB // 16, _z1, 0)

    def _zp(r, carry):
        for cb in range(8):
            packb[r, pl.ds(cb * 16, 16)] = zeros16
        return carry

    lax.fori_loop(0, HB_ROWS // 4, _zp, 0)

    def _za(r, carry):
        for cb in range(DH // 16):
            rows_a[r, pl.ds(cb * 16, 16)] = zeros16
        return carry

    lax.fori_loop(0, CHUNK, _za, 0)

    for p in range(4):
        rowids[p, pl.ds(0, 16)] = lax.iota(_i32, 16) + p * 20
        rowids[p, pl.ds(4, 16)] = lax.iota(_i32, 16) + (p * 20 + 4)

    for k in range(SHARE // CHUNK):
        pltpu.sync_copy(rows_a, agg.at[pl.ds(s * SHARE + k * CHUNK, CHUNK)])

    @pl.when(s == 0)
    def _():
        for p in range(4):
            pltpu.sync_copy(packb, hist_sh.at[pl.ds(p * 20, 20)])

    plsc.subcore_barrier()

    def _hrow(r, carry):
        for cb in range(CHUNK // 16):
            idx = dstv[r, pl.ds(cb * 16, 16)]
            plsc.addupdate_scatter(hist1d, [idx], ones16)
        return carry

    for h in range(4):
        pltpu.sync_copy(dst2d_hbm.at[pl.ds(s * CH_T + h * CH_Q, CH_Q)], dstv)
        lax.fori_loop(0, CH_Q, _hrow, 0)

    for p in range(4):
        def _pk(r, carry, p=p):
            for cb in range(8):
                packb[r, pl.ds(cb * 16, 16)] = \
                    hist1d[pl.ds((p * 20 + r) * D + cb * 16, 16)]
            return carry

        lax.fori_loop(0, HB_ROWS // 4, _pk, 0)
        pltpu.sync_copy(packb, hist_sh.at[rowids.at[p]], add=True)

    plsc.subcore_barrier()

    pltpu.sync_copy(hist_sh.at[pl.ds(s * 5, 5)], histv)
    half16 = jnp.full((16,), 0.5, _f32)
    t32 = jnp.full((16,), 1.5, _f32)
    magic = jnp.full((16,), 0x5F3759DF, _i32)
    for r in range(5):
        for cb in range(8):
            x = histv[r, pl.ds(cb * 16, 16)]
            yv = plsc.bitcast(magic - (plsc.bitcast(x, _i32) >> 1), _f32)
            for _ in range(3):
                yv = yv * (t32 - half16 * x * yv * yv)
            dv = jnp.where(x > 0, yv, 0.0)
            dinvv[r, pl.ds(cb * 16, 16)] = dv
            dinv_lv[pl.ds((r * 8 + cb) * 16, 16)] = dv

    @pl.when(c == 0)
    def _():
        pltpu.sync_copy(dinvv, dinv_hbm.at[pl.ds(s * 5, 5)])

    def _scale(y_hbm):
        def _q(q, carry):
            base = s * SHARE + q * CHUNK

            pltpu.sync_copy(y_hbm.at[pl.ds(base, CHUNK)], rows_a)

            def _r(r, carry2):
                gi = jnp.full((16,), q * CHUNK + r, _i32)
                dv = plsc.load_gather(dinv_lv, [gi])
                for cb in range(DH // 16):
                    rows_a[r, pl.ds(cb * 16, 16)] = rows_a[r, pl.ds(cb * 16, 16)] * dv
                return carry2

            lax.fori_loop(0, CHUNK, _r, 0)
            pltpu.sync_copy(rows_a, ys_sp.at[pl.ds(base, CHUNK)])
            return carry

        lax.fori_loop(0, SHARE // CHUNK, _q, 0)

    @pl.when(c == 0)
    def _():
        _scale(y0_hbm)

    @pl.when(c == 1)
    def _():
        _scale(y1_hbm)

    plsc.subcore_barrier()

    bufs = (rows_a, rows_b, rows_c, rows_d)
    gsems = (sem_a, sem_b, sem_c, sem_d)
    ssems = (sem_e, sem_f, sem_g, sem_h)

    for g in range(4):
        dumidx[pl.ds(g * 16, 16)] = lax.iota(_i32, 16) + (N_NODES + g * 16)
    for i in range(4):
        pltpu.async_copy(bufs[i], agg.at[dumidx], ssems[i], add=True)

    def _quad(k, carry):
        j0 = 4 * k
        for i in range(4):
            pltpu.make_async_copy(y0_hbm.at[pl.ds(0, CHUNK)], bufs[i],
                                  ssems[i]).wait()
        gs = [pltpu.async_copy(ys_sp.at[sidx.at[j0 + i]], bufs[i], gsems[i])
              for i in range(4)]
        for i in range(4):
            gs[i].wait()
            pltpu.async_copy(bufs[i], agg.at[dstv.at[j0 + i]], ssems[i],
                             add=True)
        return carry

    for h in range(4):
        pltpu.sync_copy(src2d_hbm.at[pl.ds(s * CH_T + h * CH_Q, CH_Q)], sidx)
        pltpu.sync_copy(dst2d_hbm.at[pl.ds(s * CH_T + h * CH_Q, CH_Q)], dstv)
        lax.fori_loop(0, CH_Q // 4, _quad, 0)

    for i in range(4):
        pltpu.make_async_copy(y0_hbm.at[pl.ds(0, CHUNK)], bufs[i],
                              ssems[i]).wait()

    plsc.subcore_barrier()

    pltpu.sync_copy(agg.at[pl.ds(s * SHARE, SHARE)],
                    aggh_hbm.at[c, pl.ds(s * SHARE, SHARE)])


_fused = functools.partial(
    pl.kernel,
    out_type=(
        jax.ShapeDtypeStruct((NC, AGG_ROWS, DH), _f32),
        jax.ShapeDtypeStruct((HB_ROWS, D), _f32),
    ),
    mesh=_mesh,
    scratch_types=[
        pltpu.VMEM((CH_Q, CHUNK), _i32),
        pltpu.VMEM((CH_Q, CHUNK), _i32),
        pltpu.VMEM((HB,), _f32),
        pltpu.VMEM((HB_ROWS // 4, D), _f32),
        pltpu.VMEM((4, 20), _i32),
        pltpu.VMEM((5, D), _f32),
        pltpu.VMEM((5, D), _f32),
        pltpu.VMEM((SHARE,), _f32),
        pltpu.VMEM((CHUNK,), _i32),
        pltpu.VMEM((CHUNK, DH), _f32),
        pltpu.VMEM((CHUNK, DH), _f32),
        pltpu.VMEM((CHUNK, DH), _f32),
        pltpu.VMEM((CHUNK, DH), _f32),
        pltpu.SemaphoreType.DMA,
        pltpu.SemaphoreType.DMA,
        pltpu.SemaphoreType.DMA,
        pltpu.SemaphoreType.DMA,
        pltpu.SemaphoreType.DMA,
        pltpu.SemaphoreType.DMA,
        pltpu.SemaphoreType.DMA,
        pltpu.SemaphoreType.DMA,
        pltpu.VMEM_SHARED((HB_ROWS, D), _f32),
        pltpu.VMEM_SHARED((AGG_ROWS, DH), _f32),
        pltpu.VMEM_SHARED((AGG_ROWS, DH), _f32),
    ],
    compiler_params=pltpu.CompilerParams(needs_layout_passes=False,
                                         use_tc_tiling_on_sc=False),
)(_fused_body)


def _comb_body(coef_ref, y_ref, x_ref, dinv_ref, ap_ref, out_ref):
    c0, c1, c2 = coef_ref[0], coef_ref[1], coef_ref[2]
    dinv = dinv_ref[...]
    out_ref[:, :DH] = (c0 * y_ref[:, :DH]
                       + c1 * (dinv * ap_ref[0])
                       + c2 * x_ref[:, :DH])
    out_ref[:, DH:] = (c0 * y_ref[:, DH:]
                       + c1 * (dinv * ap_ref[1])
                       + c2 * x_ref[:, DH:])


_comb = pl.pallas_call(
    _comb_body,
    grid=(GRID,),
    in_specs=[
        pl.BlockSpec(memory_space=pltpu.MemorySpace.SMEM),
        pl.BlockSpec((RB, D), lambda i: (i, 0)),
        pl.BlockSpec((RB, D), lambda i: (i, 0)),
        pl.BlockSpec((RB, 1), lambda i: (i, 0)),
        pl.BlockSpec((2, RB, DH), lambda i: (0, i, 0)),
    ],
    out_specs=pl.BlockSpec((RB, D), lambda i: (i, 0)),
    out_shape=jax.ShapeDtypeStruct((N_NODES, D), _f32),
)


def kernel(edge_index, Y, X, alp, lam):
    src = edge_index[0]
    dst = edge_index[1]

    npad = E_PAD - N_EDGES
    ar = jnp.arange(npad, dtype=_i32)
    src_p = jnp.concatenate([src, ar % N_NODES]).reshape(N_CHUNK_ROWS, CHUNK)
    dst_p = jnp.concatenate([dst, N_NODES + ar % (AGG_ROWS - N_NODES)]
                            ).reshape(N_CHUNK_ROWS, CHUNK)

    zpad = jnp.zeros((AGG_ROWS - N_NODES, DH), _f32)
    y0 = jnp.concatenate([Y[:, :DH], zpad])
    y1 = jnp.concatenate([Y[:, DH:], zpad])

    aggh, dinv2d = _fused(dst_p, src_p, y0, y1)
    dinv_col = dinv2d.reshape(HB)[:N_NODES].reshape(N_NODES, 1)

    al = alp * lam
    coef = jnp.stack([1.0 - al - alp, al, alp]).astype(_f32)
    return _comb(coef, Y, X, dinv_col, aggh)
